# baseline (speedup 1.0000x reference)
import jax
import jax.numpy as jnp
from jax.experimental import pallas as pl
from jax.experimental.pallas import tpu as pltpu

_TM = 2048


def _fused_body(x_ref, lab_ref, w_ref, b_ref, o_ref, lab_o_ref):
    o_ref[...] = (
        jnp.dot(x_ref[...], w_ref[...], preferred_element_type=jnp.float32)
        + b_ref[...]
    )

    @pl.when(pl.program_id(0) == 0)
    def _():
        lab_o_ref[...] = lab_ref[...].astype(jnp.float32)


def _mlp_body(x_ref, w_ref, b_ref, o_ref):
    o_ref[...] = (
        jnp.dot(x_ref[...], w_ref[...], preferred_element_type=jnp.float32)
        + b_ref[...]
    )


def kernel(pooled_features, labels, weight, bias):
    b, g, d = pooled_features.shape
    out = weight.shape[1]
    n = b * g

    flat = pooled_features.reshape(n, d).astype(jnp.float32)
    w = weight.astype(jnp.float32)
    bias2d = bias.astype(jnp.float32).reshape(1, out)

    tm = min(_TM, n)
    if n % tm == 0:
        grid = (n // tm,)
        feats, label = pl.pallas_call(
            _fused_body,
            out_shape=(
                jax.ShapeDtypeStruct((n, out), jnp.float32),
                jax.ShapeDtypeStruct((b, g), jnp.float32),
            ),
            grid=grid,
            in_specs=[
                pl.BlockSpec((tm, d), lambda i: (i, 0)),
                pl.BlockSpec((b, g), lambda i: (0, 0)),
                pl.BlockSpec((d, out), lambda i: (0, 0)),
                pl.BlockSpec((1, out), lambda i: (0, 0)),
            ],
            out_specs=(
                pl.BlockSpec((tm, out), lambda i: (i, 0)),
                pl.BlockSpec((b, g), lambda i: (0, 0)),
            ),
            compiler_params=pltpu.CompilerParams(
                dimension_semantics=("arbitrary",),
            ),
        )(flat, labels, w, bias2d)
        return feats, label

    grid = (pl.cdiv(n, tm),)
    feats = pl.pallas_call(
        _mlp_body,
        out_shape=jax.ShapeDtypeStruct((n, out), jnp.float32),
        grid=grid,
        in_specs=[
            pl.BlockSpec((tm, d), lambda i: (i, 0)),
            pl.BlockSpec((d, out), lambda i: (0, 0)),
            pl.BlockSpec((1, out), lambda i: (0, 0)),
        ],
        out_specs=pl.BlockSpec((tm, out), lambda i: (i, 0)),
        compiler_params=pltpu.CompilerParams(
            dimension_semantics=("parallel",),
        ),
    )(flat, w, bias2d)
    return feats, labels.astype(jnp.float32)


# speedup vs baseline: 1.1478x; 1.1478x over previous
import jax
import jax.numpy as jnp
from jax.experimental import pallas as pl
from jax.experimental.pallas import tpu as pltpu

_TM = 2048


def _fused_body(x_ref, lab_ref, w_ref, b_ref, o_ref, lab_o_ref):
    o_ref[...] = (
        jnp.dot(x_ref[...], w_ref[...], preferred_element_type=jnp.float32)
        + b_ref[...]
    )

    @pl.when(pl.program_id(0) == 0)
    def _():
        lab_o_ref[...] = lab_ref[...].astype(jnp.float32)


def _mlp_body(x_ref, w_ref, b_ref, o_ref):
    o_ref[...] = (
        jnp.dot(x_ref[...], w_ref[...], preferred_element_type=jnp.float32)
        + b_ref[...]
    )


def kernel(pooled_features, labels, weight, bias):
    b, g, d = pooled_features.shape
    out = weight.shape[1]
    n = b * g

    flat = pooled_features.reshape(n, d).astype(jnp.float32)
    w = weight.astype(jnp.float32)
    bias2d = bias.astype(jnp.float32).reshape(1, out)

    tm = min(_TM, n)
    if False:
        grid = (n // tm,)
        feats, label = pl.pallas_call(
            _fused_body,
            out_shape=(
                jax.ShapeDtypeStruct((n, out), jnp.float32),
                jax.ShapeDtypeStruct((b, g), jnp.float32),
            ),
            grid=grid,
            in_specs=[
                pl.BlockSpec((tm, d), lambda i: (i, 0)),
                pl.BlockSpec((b, g), lambda i: (0, 0)),
                pl.BlockSpec((d, out), lambda i: (0, 0)),
                pl.BlockSpec((1, out), lambda i: (0, 0)),
            ],
            out_specs=(
                pl.BlockSpec((tm, out), lambda i: (i, 0)),
                pl.BlockSpec((b, g), lambda i: (0, 0)),
            ),
            compiler_params=pltpu.CompilerParams(
                dimension_semantics=("arbitrary",),
            ),
        )(flat, labels, w, bias2d)
        return feats, label

    grid = (pl.cdiv(n, tm),)
    feats = pl.pallas_call(
        _mlp_body,
        out_shape=jax.ShapeDtypeStruct((n, out), jnp.float32),
        grid=grid,
        in_specs=[
            pl.BlockSpec((tm, d), lambda i: (i, 0)),
            pl.BlockSpec((d, out), lambda i: (0, 0)),
            pl.BlockSpec((1, out), lambda i: (0, 0)),
        ],
        out_specs=pl.BlockSpec((tm, out), lambda i: (i, 0)),
        compiler_params=pltpu.CompilerParams(
            dimension_semantics=("parallel",),
        ),
    )(flat, w, bias2d)
    return feats, labels.astype(jnp.float32)


# speedup vs baseline: 1.1516x; 1.0034x over previous
import jax
import jax.numpy as jnp
from jax.experimental import pallas as pl
from jax.experimental.pallas import tpu as pltpu

_TM = 4096


def _fused_body(x_ref, lab_ref, w_ref, b_ref, o_ref, lab_o_ref):
    o_ref[...] = (
        jnp.dot(x_ref[...], w_ref[...], preferred_element_type=jnp.float32)
        + b_ref[...]
    )

    @pl.when(pl.program_id(0) == 0)
    def _():
        lab_o_ref[...] = lab_ref[...].astype(jnp.float32)


def _mlp_body(x_ref, w_ref, b_ref, o_ref):
    o_ref[...] = (
        jnp.dot(x_ref[...], w_ref[...], preferred_element_type=jnp.float32)
        + b_ref[...]
    )


def kernel(pooled_features, labels, weight, bias):
    b, g, d = pooled_features.shape
    out = weight.shape[1]
    n = b * g

    flat = pooled_features.reshape(n, d).astype(jnp.float32)
    w = weight.astype(jnp.float32)
    bias2d = bias.astype(jnp.float32).reshape(1, out)

    tm = min(_TM, n)
    if False:
        grid = (n // tm,)
        feats, label = pl.pallas_call(
            _fused_body,
            out_shape=(
                jax.ShapeDtypeStruct((n, out), jnp.float32),
                jax.ShapeDtypeStruct((b, g), jnp.float32),
            ),
            grid=grid,
            in_specs=[
                pl.BlockSpec((tm, d), lambda i: (i, 0)),
                pl.BlockSpec((b, g), lambda i: (0, 0)),
                pl.BlockSpec((d, out), lambda i: (0, 0)),
                pl.BlockSpec((1, out), lambda i: (0, 0)),
            ],
            out_specs=(
                pl.BlockSpec((tm, out), lambda i: (i, 0)),
                pl.BlockSpec((b, g), lambda i: (0, 0)),
            ),
            compiler_params=pltpu.CompilerParams(
                dimension_semantics=("arbitrary",),
            ),
        )(flat, labels, w, bias2d)
        return feats, label

    grid = (pl.cdiv(n, tm),)
    feats = pl.pallas_call(
        _mlp_body,
        out_shape=jax.ShapeDtypeStruct((n, out), jnp.float32),
        grid=grid,
        in_specs=[
            pl.BlockSpec((tm, d), lambda i: (i, 0)),
            pl.BlockSpec((d, out), lambda i: (0, 0)),
            pl.BlockSpec((1, out), lambda i: (0, 0)),
        ],
        out_specs=pl.BlockSpec((tm, out), lambda i: (i, 0)),
        compiler_params=pltpu.CompilerParams(
            dimension_semantics=("parallel",),
        ),
    )(flat, w, bias2d)
    return feats, labels.astype(jnp.float32)


# speedup vs baseline: 1.5436x; 1.3403x over previous
import jax
import jax.numpy as jnp
from jax.experimental import pallas as pl
from jax.experimental.pallas import tpu as pltpu

_TM = 2048


def _mlp_t_body(x_ref, w_ref, b_ref, ot_ref):
    ot_ref[...] = (
        jax.lax.dot_general(
            w_ref[...], x_ref[...],
            dimension_numbers=(((0,), (1,)), ((), ())),
            preferred_element_type=jnp.float32,
        )
        + b_ref[...]
    )


def kernel(pooled_features, labels, weight, bias):
    b, g, d = pooled_features.shape
    out = weight.shape[1]
    n = b * g

    flat = pooled_features.reshape(n, d).astype(jnp.float32)
    w = weight.astype(jnp.float32)
    bias_col = bias.astype(jnp.float32).reshape(out, 1)

    tm = min(_TM, n)
    grid = (pl.cdiv(n, tm),)
    feats_t = pl.pallas_call(
        _mlp_t_body,
        out_shape=jax.ShapeDtypeStruct((out, n), jnp.float32),
        grid=grid,
        in_specs=[
            pl.BlockSpec((tm, d), lambda i: (i, 0)),
            pl.BlockSpec((d, out), lambda i: (0, 0)),
            pl.BlockSpec((out, 1), lambda i: (0, 0)),
        ],
        out_specs=pl.BlockSpec((out, tm), lambda i: (0, i)),
        compiler_params=pltpu.CompilerParams(
            dimension_semantics=("parallel",),
        ),
    )(flat, w, bias_col)

    return feats_t.T, labels.astype(jnp.float32)


# speedup vs baseline: 1.5904x; 1.0303x over previous
import jax
import jax.numpy as jnp
from jax.experimental import pallas as pl
from jax.experimental.pallas import tpu as pltpu

_TM = 2048


def _fused_t_body(x_ref, labt_ref, w_ref, b_ref, ot_ref, labt_o_ref):
    ot_ref[...] = (
        jax.lax.dot_general(
            w_ref[...], x_ref[...],
            dimension_numbers=(((0,), (1,)), ((), ())),
            preferred_element_type=jnp.float32,
        )
        + b_ref[...]
    )

    @pl.when(pl.program_id(0) == 0)
    def _():
        labt_o_ref[...] = labt_ref[...].astype(jnp.float32)


def _mlp_t_body(x_ref, w_ref, b_ref, ot_ref):
    ot_ref[...] = (
        jax.lax.dot_general(
            w_ref[...], x_ref[...],
            dimension_numbers=(((0,), (1,)), ((), ())),
            preferred_element_type=jnp.float32,
        )
        + b_ref[...]
    )


def kernel(pooled_features, labels, weight, bias):
    b, g, d = pooled_features.shape
    out = weight.shape[1]
    n = b * g

    flat = pooled_features.reshape(n, d).astype(jnp.float32)
    w = weight.astype(jnp.float32)
    bias_col = bias.astype(jnp.float32).reshape(out, 1)

    tm = min(_TM, n)
    grid = (pl.cdiv(n, tm),)

    if n % tm == 0:
        feats_t, labt = pl.pallas_call(
            _fused_t_body,
            out_shape=(
                jax.ShapeDtypeStruct((out, n), jnp.float32),
                jax.ShapeDtypeStruct((g, b), jnp.float32),
            ),
            grid=grid,
            in_specs=[
                pl.BlockSpec((tm, d), lambda i: (i, 0)),
                pl.BlockSpec((g, b), lambda i: (0, 0)),
                pl.BlockSpec((d, out), lambda i: (0, 0)),
                pl.BlockSpec((out, 1), lambda i: (0, 0)),
            ],
            out_specs=(
                pl.BlockSpec((out, tm), lambda i: (0, i)),
                pl.BlockSpec((g, b), lambda i: (0, 0)),
            ),
            compiler_params=pltpu.CompilerParams(
                dimension_semantics=("arbitrary",),
            ),
        )(flat, labels.T, w, bias_col)
        return feats_t.T, labt.T

    feats_t = pl.pallas_call(
        _mlp_t_body,
        out_shape=jax.ShapeDtypeStruct((out, n), jnp.float32),
        grid=grid,
        in_specs=[
            pl.BlockSpec((tm, d), lambda i: (i, 0)),
            pl.BlockSpec((d, out), lambda i: (0, 0)),
            pl.BlockSpec((out, 1), lambda i: (0, 0)),
        ],
        out_specs=pl.BlockSpec((out, tm), lambda i: (0, i)),
        compiler_params=pltpu.CompilerParams(
            dimension_semantics=("parallel",),
        ),
    )(flat, w, bias_col)
    return feats_t.T, labels.astype(jnp.float32)


# speedup vs baseline: 1.6668x; 1.0481x over previous
import jax
import jax.numpy as jnp
from jax.experimental import pallas as pl
from jax.experimental.pallas import tpu as pltpu

_TM = 2048


def _fused_t_body(x_ref, labt_ref, wbt_ref, ot_ref, labt_o_ref):
    d = x_ref.shape[1]
    ot_ref[...] = (
        jax.lax.dot_general(
            wbt_ref[:, :d], x_ref[...],
            dimension_numbers=(((1,), (1,)), ((), ())),
            preferred_element_type=jnp.float32,
        )
        + wbt_ref[:, d:d + 1]
    )

    @pl.when(pl.program_id(0) == 0)
    def _():
        labt_o_ref[...] = labt_ref[...].astype(jnp.float32)


def _mlp_t_body(x_ref, w_ref, b_ref, ot_ref):
    ot_ref[...] = (
        jax.lax.dot_general(
            w_ref[...], x_ref[...],
            dimension_numbers=(((0,), (1,)), ((), ())),
            preferred_element_type=jnp.float32,
        )
        + b_ref[...]
    )


def kernel(pooled_features, labels, weight, bias):
    b, g, d = pooled_features.shape
    out = weight.shape[1]
    n = b * g

    flat = pooled_features.reshape(n, d).astype(jnp.float32)
    w = weight.astype(jnp.float32)
    bias_col = bias.astype(jnp.float32).reshape(out, 1)

    tm = min(_TM, n)
    grid = (pl.cdiv(n, tm),)

    if n % tm == 0:
        wbt = jnp.concatenate([w.T, bias_col], axis=1)
        feats_t, labt = pl.pallas_call(
            _fused_t_body,
            out_shape=(
                jax.ShapeDtypeStruct((out, n), jnp.float32),
                jax.ShapeDtypeStruct((g, b), jnp.float32),
            ),
            grid=grid,
            in_specs=[
                pl.BlockSpec((tm, d), lambda i: (i, 0)),
                pl.BlockSpec((g, b), lambda i: (0, 0)),
                pl.BlockSpec((out, d + 1), lambda i: (0, 0)),
            ],
            out_specs=(
                pl.BlockSpec((out, tm), lambda i: (0, i)),
                pl.BlockSpec((g, b), lambda i: (0, 0)),
            ),
            compiler_params=pltpu.CompilerParams(
                dimension_semantics=("arbitrary",),
            ),
        )(flat, labels.T, wbt)
        return feats_t.T, labt.T

    feats_t = pl.pallas_call(
        _mlp_t_body,
        out_shape=jax.ShapeDtypeStruct((out, n), jnp.float32),
        grid=grid,
        in_specs=[
            pl.BlockSpec((tm, d), lambda i: (i, 0)),
            pl.BlockSpec((d, out), lambda i: (0, 0)),
            pl.BlockSpec((out, 1), lambda i: (0, 0)),
        ],
        out_specs=pl.BlockSpec((out, tm), lambda i: (0, i)),
        compiler_params=pltpu.CompilerParams(
            dimension_semantics=("parallel",),
        ),
    )(flat, w, bias_col)
    return feats_t.T, labels.astype(jnp.float32)


# speedup vs baseline: 1.6740x; 1.0043x over previous
import jax
import jax.numpy as jnp
from jax.experimental import pallas as pl
from jax.experimental.pallas import tpu as pltpu

_TM = 2048


def _fused_t_body(x_ref, labt_ref, wbt_ref, ot_ref, labt_o_ref):
    d = x_ref.shape[1]
    ot_ref[...] = (
        jax.lax.dot_general(
            wbt_ref[:, :d], x_ref[...],
            dimension_numbers=(((1,), (1,)), ((), ())),
            preferred_element_type=jnp.float32,
        )
        + wbt_ref[:, d:d + 1]
    )

    @pl.when(pl.program_id(0) == 0)
    def _():
        labt_o_ref[...] = labt_ref[...].astype(jnp.float32)


def _mlp_t_body(x_ref, w_ref, b_ref, ot_ref):
    ot_ref[...] = (
        jax.lax.dot_general(
            w_ref[...], x_ref[...],
            dimension_numbers=(((0,), (1,)), ((), ())),
            preferred_element_type=jnp.float32,
        )
        + b_ref[...]
    )


def kernel(pooled_features, labels, weight, bias):
    b, g, d = pooled_features.shape
    out = weight.shape[1]
    n = b * g

    flat = pooled_features.reshape(n, d).astype(jnp.float32)
    w = weight.astype(jnp.float32)
    bias_col = bias.astype(jnp.float32).reshape(out, 1)

    tm = min(_TM, n)
    grid = (pl.cdiv(n, tm),)

    if n % tm == 0:
        wbt = jnp.concatenate([w, bias_col.T], axis=0).T
        feats_t, labt = pl.pallas_call(
            _fused_t_body,
            out_shape=(
                jax.ShapeDtypeStruct((out, n), jnp.float32),
                jax.ShapeDtypeStruct((g, b), jnp.float32),
            ),
            grid=grid,
            in_specs=[
                pl.BlockSpec((tm, d), lambda i: (i, 0)),
                pl.BlockSpec((g, b), lambda i: (0, 0)),
                pl.BlockSpec((out, d + 1), lambda i: (0, 0)),
            ],
            out_specs=(
                pl.BlockSpec((out, tm), lambda i: (0, i)),
                pl.BlockSpec((g, b), lambda i: (0, 0)),
            ),
            compiler_params=pltpu.CompilerParams(
                dimension_semantics=("arbitrary",),
            ),
        )(flat, labels.T, wbt)
        return feats_t.T, labt.T

    feats_t = pl.pallas_call(
        _mlp_t_body,
        out_shape=jax.ShapeDtypeStruct((out, n), jnp.float32),
        grid=grid,
        in_specs=[
            pl.BlockSpec((tm, d), lambda i: (i, 0)),
            pl.BlockSpec((d, out), lambda i: (0, 0)),
            pl.BlockSpec((out, 1), lambda i: (0, 0)),
        ],
        out_specs=pl.BlockSpec((out, tm), lambda i: (0, i)),
        compiler_params=pltpu.CompilerParams(
            dimension_semantics=("parallel",),
        ),
    )(flat, w, bias_col)
    return feats_t.T, labels.astype(jnp.float32)
